# baseline (speedup 1.0000x reference)
import jax
import jax.numpy as jnp
from jax import lax
from jax.experimental import pallas as pl
from jax.experimental.pallas import tpu as pltpu
from jax.experimental.pallas import tpu_sc as plsc

N = 1_000_000
NC = 2
NS = 16
NW = NC * NS
L = 16
G = 8

CHUNK = (N // NW) // (G * L) * (G * L)
TAIL = N - CHUNK * NW
TBL = 18
FB = 65536


def _gather_group(z_ref, p_ref, scale_v, shift_v, off, n_chains):
    zs = [z_ref[pl.ds(off + k * L, L)] for k in range(n_chains)]
    svs = [plsc.load_gather(scale_v, [zv]) for zv in zs]
    tvs = [plsc.load_gather(shift_v, [zv]) for zv in zs]
    for k in range(n_chains):
        packed = plsc.pack(svs[k], tvs[k], format=plsc.PackFormat.INTERLEAVED)
        p_ref[pl.ds(off + k * L, L)] = plsc.bitcast(packed, jnp.uint32)


def _gather_body(z_hbm, scale_hbm, shift_hbm, p_hbm,
                 z_v, p_v, scale_v, shift_v):
    wid = lax.axis_index("s") * NC + lax.axis_index("c")
    base = wid * CHUNK

    pltpu.sync_copy(scale_hbm, scale_v)
    pltpu.sync_copy(shift_hbm, shift_v)

    pltpu.sync_copy(z_hbm.at[pl.ds(base, CHUNK)], z_v)

    def body(i, _):
        _gather_group(z_v, p_v, scale_v, shift_v, i * (G * L), G)
        return _

    lax.fori_loop(0, CHUNK // (G * L), body, None)

    pltpu.sync_copy(p_v, p_hbm.at[pl.ds(base, CHUNK)])

    @pl.when(wid == NW - 1)
    def _():
        tbase = CHUNK * NW
        pltpu.sync_copy(z_hbm.at[pl.ds(tbase, TAIL)], z_v.at[pl.ds(0, TAIL)])

        def tbody(i, _):
            _gather_group(z_v, p_v, scale_v, shift_v, i * (G * L), G)
            return _

        lax.fori_loop(0, TAIL // (G * L), tbody, None)
        for j in range(TAIL // (G * L) * G, TAIL // L):
            _gather_group(z_v, p_v, scale_v, shift_v, j * L, 1)
        pltpu.sync_copy(p_v.at[pl.ds(0, TAIL)], p_hbm.at[pl.ds(tbase, TAIL)])


@jax.jit
def kernel(inputs, z, scale_table, shift_table):
    zi = z.astype(jnp.int32)
    scale = scale_table.reshape(TBL)
    shift = shift_table.reshape(TBL)

    mesh = plsc.VectorSubcoreMesh(core_axis_name="c", subcore_axis_name="s",
                                  num_cores=NC, num_subcores=NS)
    p = pl.kernel(
        _gather_body,
        out_type=jax.ShapeDtypeStruct((N,), jnp.uint32),
        mesh=mesh,
        compiler_params=pltpu.CompilerParams(needs_layout_passes=False),
        scratch_types=[
            pltpu.VMEM((CHUNK,), jnp.int32),
            pltpu.VMEM((CHUNK,), jnp.uint32),
            pltpu.VMEM((TBL,), jnp.float32),
            pltpu.VMEM((TBL,), jnp.float32),
        ],
    )(zi, scale, shift)
    def _fma_body(x_ref, p_ref, o_ref):
        pv = p_ref[...]
        sv = jax.lax.bitcast_convert_type(pv << 16, jnp.float32)
        tv = jax.lax.bitcast_convert_type(pv & jnp.uint32(0xFFFF0000),
                                          jnp.float32)
        o_ref[...] = x_ref[...] * sv + tv

    out = pl.pallas_call(
        _fma_body,
        grid=(pl.cdiv(N, FB),),
        in_specs=[pl.BlockSpec((FB,), lambda i: (i,)),
                  pl.BlockSpec((FB,), lambda i: (i,))],
        out_specs=pl.BlockSpec((FB,), lambda i: (i,)),
        out_shape=jax.ShapeDtypeStruct((N,), jnp.float32),
    )(inputs.reshape(N), p)
    return out.reshape(N, 1)


# speedup vs baseline: 1.7674x; 1.7674x over previous
import jax
import jax.numpy as jnp
from jax import lax
from jax.experimental import pallas as pl
from jax.experimental.pallas import tpu as pltpu
from jax.experimental.pallas import tpu_sc as plsc

N = 1_000_000
NC = 2
NS = 16
NW = NC * NS
L = 16
G = 8

CHUNK = (N // NW) // (G * L) * (G * L)
TAIL = N - CHUNK * NW
TBL = 18


def _gather_group(z_ref, s_ref, t_ref, scale_v, shift_v, off, n_chains):
    zs = [z_ref[pl.ds(off + k * L, L)] for k in range(n_chains)]
    svs = [plsc.load_gather(scale_v, [zv]) for zv in zs]
    tvs = [plsc.load_gather(shift_v, [zv]) for zv in zs]
    for k in range(n_chains):
        s_ref[pl.ds(off + k * L, L)] = svs[k]
        t_ref[pl.ds(off + k * L, L)] = tvs[k]


def _gather_body(z_hbm, scale_hbm, shift_hbm, s_hbm, t_hbm,
                 z_v, s_v, t_v, scale_v, shift_v):
    wid = lax.axis_index("s") * NC + lax.axis_index("c")
    base = wid * CHUNK

    pltpu.sync_copy(scale_hbm, scale_v)
    pltpu.sync_copy(shift_hbm, shift_v)

    pltpu.sync_copy(z_hbm.at[pl.ds(base, CHUNK)], z_v)

    def body(i, _):
        _gather_group(z_v, s_v, t_v, scale_v, shift_v, i * (G * L), G)
        return _

    lax.fori_loop(0, CHUNK // (G * L), body, None)

    pltpu.sync_copy(s_v, s_hbm.at[pl.ds(base, CHUNK)])
    pltpu.sync_copy(t_v, t_hbm.at[pl.ds(base, CHUNK)])

    @pl.when(wid == NW - 1)
    def _():
        tbase = CHUNK * NW
        pltpu.sync_copy(z_hbm.at[pl.ds(tbase, TAIL)], z_v.at[pl.ds(0, TAIL)])

        def tbody(i, _):
            _gather_group(z_v, s_v, t_v, scale_v, shift_v, i * (G * L), G)
            return _

        lax.fori_loop(0, TAIL // (G * L), tbody, None)
        for j in range(TAIL // (G * L) * G, TAIL // L):
            _gather_group(z_v, s_v, t_v, scale_v, shift_v, j * L, 1)
        pltpu.sync_copy(s_v.at[pl.ds(0, TAIL)], s_hbm.at[pl.ds(tbase, TAIL)])
        pltpu.sync_copy(t_v.at[pl.ds(0, TAIL)], t_hbm.at[pl.ds(tbase, TAIL)])


@jax.jit
def kernel(inputs, z, scale_table, shift_table):
    zi = z.astype(jnp.int32)
    scale = scale_table.reshape(TBL)
    shift = shift_table.reshape(TBL)

    mesh = plsc.VectorSubcoreMesh(core_axis_name="c", subcore_axis_name="s",
                                  num_cores=NC, num_subcores=NS)
    s, t = pl.kernel(
        _gather_body,
        out_type=(jax.ShapeDtypeStruct((N,), jnp.float32),
                  jax.ShapeDtypeStruct((N,), jnp.float32)),
        mesh=mesh,
        compiler_params=pltpu.CompilerParams(needs_layout_passes=False),
        scratch_types=[
            pltpu.VMEM((CHUNK,), jnp.int32),
            pltpu.VMEM((CHUNK,), jnp.float32),
            pltpu.VMEM((CHUNK,), jnp.float32),
            pltpu.VMEM((TBL,), jnp.float32),
            pltpu.VMEM((TBL,), jnp.float32),
        ],
    )(zi, scale, shift)
    return inputs * s[:, None] + t[:, None]
